# baseline (speedup 1.0000x reference)
import functools

import numpy as np
import jax
import jax.numpy as jnp
from jax import lax
from jax.experimental import pallas as pl
from jax.experimental.pallas import tpu as pltpu
from jax.experimental.pallas import tpu_sc as plsc

_NC = 2
_NS = 16
_NW = _NC * _NS
_LANES = 16
_NBUF = 4
_CH = 40


def _pos_encoding(dk: int, length: int) -> np.ndarray:
    i = np.arange(dk)
    even = (i % 2 == 0).astype(np.float64)
    power = 10000.0 ** (2.0 * i / dk)
    pos = np.arange(length)[:, None]
    ang = pos / power[None, :]
    return (np.sin(ang) * even + np.cos(ang) * (1.0 - even)).astype(np.float32)


@functools.partial(jax.jit, static_argnames=("interpret",))
def kernel(tokens, table, *, interpret=False):
    B, L = tokens.shape
    V, D = table.shape
    assert B % _NW == 0 and D % _LANES == 0 and V % 2 == 0
    spw = B // _NW
    ch = _CH
    assert L % ch == 0
    nch = L // ch
    nk = spw * nch
    assert nk % _NBUF == 0
    gfull, gtail = ch // _LANES, ch % _LANES
    chp = (gfull + (1 if gtail else 0)) * _LANES

    pos = jnp.asarray(_pos_encoding(D, L))
    tok = tokens.astype(jnp.int32)
    idx2 = (tok >> 1).reshape(_NW, nk, ch)
    par = jnp.pad(
        (tok & 1).reshape(_NW, nk, ch), ((0, 0), (0, 0), (0, chp - ch))
    )
    table2 = table.reshape(V // 2, 2 * D)

    mesh = plsc.VectorSubcoreMesh(
        core_axis_name="c", subcore_axis_name="s",
        num_cores=_NC, num_subcores=_NS,
    )

    @functools.partial(
        pl.kernel,
        out_type=jax.ShapeDtypeStruct((B, L, D), jnp.float32),
        mesh=mesh,
        scratch_types=[
            pltpu.VMEM((nk, ch), jnp.int32),
            pltpu.VMEM((nk, chp), jnp.int32),
            pltpu.VMEM((_NBUF, ch, 2 * D), jnp.float32),
            pltpu.VMEM((_NBUF, ch, D), jnp.float32),
            pltpu.VMEM((L, D), jnp.float32),
            [pltpu.SemaphoreType.DMA for _ in range(_NBUF)],
        ],
        compiler_params=pltpu.CompilerParams(use_tc_tiling_on_sc=True),
        interpret=interpret,
    )
    def emb_kernel(idx_hbm, par_hbm, pos_hbm, table_hbm, out_hbm,
                   idx_all, par_all, rows, rows_o, pos_v, sem_g):
        wid = lax.axis_index("s") * _NC + lax.axis_index("c")
        base = wid * spw
        pltpu.sync_copy(pos_hbm, pos_v)
        pltpu.sync_copy(idx_hbm.at[wid], idx_all)
        pltpu.sync_copy(par_hbm.at[wid], par_all)

        def gather(k, b, issue):
            mk = pltpu.async_copy if issue else pltpu.make_async_copy
            return mk(table_hbm.at[idx_all.at[k]], rows.at[b], sem_g[b])

        for p in range(_NBUF - 1):
            gather(p, p, True)

        @pl.loop(0, nk, step=_NBUF)
        def _outer(ko):
            for b in range(_NBUF):
                k = ko + b
                gather(k, b, False).wait()
                j = k // nch
                h = k % nch
                r0 = h * ch

                def do_row(rr, off):
                    for c in range(D // _LANES):
                        rows_o[b, rr, pl.ds(c * _LANES, _LANES)] = (
                            rows[b, rr, pl.ds(off + c * _LANES, _LANES)]
                            + pos_v[r0 + rr, pl.ds(c * _LANES, _LANES)]
                        )

                for g in range(gfull):
                    par16 = par_all[k, pl.ds(g * _LANES, _LANES)]
                    for i in range(_LANES):
                        do_row(g * _LANES + i, par16[i] * D)
                if gtail:
                    par16 = par_all[k, pl.ds(gfull * _LANES, _LANES)]
                    for i in range(gtail):
                        do_row(gfull * _LANES + i, par16[i] * D)

                bp = (b - 1) % _NBUF

                @pl.when(k + _NBUF - 1 < nk)
                def _():
                    gather(k + _NBUF - 1, bp, True)

                pltpu.sync_copy(
                    rows_o.at[b],
                    out_hbm.at[base + j, pl.ds(r0, ch)],
                )

    return emb_kernel(idx2, par, pos, table2)
